# baseline (speedup 1.0000x reference)
import functools

import jax
import jax.numpy as jnp
from jax import lax
from jax.experimental import pallas as pl
from jax.experimental.pallas import tpu as pltpu
from jax.experimental.pallas import tpu_sc as plsc

_N_TIMES = 288
_B = 16384
_D_LINK = 32
_D_TIME = 8

_NC = 2
_NS = 16
_NW = _NC * _NS
_CHUNK = _B // _NW
_HALF = _CHUNK // 2
_L = 16

_sc_mesh = plsc.VectorSubcoreMesh(core_axis_name="c", subcore_axis_name="s")


@functools.partial(
    pl.kernel,
    mesh=_sc_mesh,
    compiler_params=pltpu.CompilerParams(
        use_tc_tiling_on_sc=False, needs_layout_passes=False),
    out_type=[
        jax.ShapeDtypeStruct((_B, _D_LINK), jnp.float32),
        jax.ShapeDtypeStruct((_B,), jnp.float32),
    ],
    scratch_types=[
        pltpu.VMEM((_CHUNK,), jnp.int32),
        pltpu.VMEM((_CHUNK,), jnp.int32),
        pltpu.VMEM((2, _HALF), jnp.int32),
        pltpu.VMEM((_CHUNK,), jnp.int32),
        pltpu.VMEM((2, _HALF), jnp.int32),
        pltpu.VMEM((_CHUNK,), jnp.int32),
        pltpu.VMEM((_HALF, 128), jnp.float32),
        pltpu.VMEM((_HALF, 128), jnp.float32),
        pltpu.VMEM((_CHUNK, _D_LINK), jnp.float32),
        pltpu.VMEM((_CHUNK,), jnp.float32),
        pltpu.SemaphoreType.DMA,
        pltpu.SemaphoreType.DMA,
    ],
)
def _sc_gather(link_idx_hbm, time_idx_hbm, link_tab_hbm, cross_tab_hbm,
               le_out, cr_out,
               li_v, ti_v, lrow_v, loff_v, crow_v, ccol_v,
               lrows_v, crows_v, le_v, cr_v, sem_l, sem_c):
    wid = lax.axis_index("s") * _NC + lax.axis_index("c")
    base = wid * _CHUNK
    pltpu.sync_copy(link_idx_hbm.at[pl.ds(base, _CHUNK)], li_v)
    pltpu.sync_copy(time_idx_hbm.at[pl.ds(base, _CHUNK)], ti_v)

    for k in range(_CHUNK // _L):
        a = li_v[pl.ds(k * _L, _L)]
        b = ti_v[pl.ds(k * _L, _L)]
        lrow_v[k * _L // _HALF, pl.ds(k * _L % _HALF, _L)] = (
            lax.shift_right_logical(a, 2))
        loff_v[pl.ds(k * _L, _L)] = lax.shift_left(
            lax.bitwise_and(a, 3), 5)
        ci = a * _N_TIMES + b
        crow_v[k * _L // _HALF, pl.ds(k * _L % _HALF, _L)] = (
            lax.shift_right_logical(ci, 7))
        ccol_v[pl.ds(k * _L, _L)] = lax.bitwise_and(ci, 127)

    for h in range(2):
        cl = pltpu.async_copy(link_tab_hbm.at[lrow_v.at[h]], lrows_v, sem_l)
        cc = pltpu.async_copy(cross_tab_hbm.at[crow_v.at[h]], crows_v, sem_c)
        cl.wait()

        def _extract(c, carry, h=h):
            offv = loff_v[pl.ds(h * _HALF + c * _L, _L)]
            for j in range(_L):
                o = offv[j]
                i = c * _L + j
                r = h * _HALF + i
                le_v[r, pl.ds(0, _L)] = lrows_v[i, pl.ds(o, _L)]
                le_v[r, pl.ds(_L, _L)] = lrows_v[i, pl.ds(o + _L, _L)]
            return carry

        lax.fori_loop(0, _HALF // _L, _extract, 0)

        cc.wait()
        for k in range(_HALF // _L):
            rid = lax.broadcasted_iota(jnp.int32, (_L,), 0) + k * _L
            cid = ccol_v[pl.ds(h * _HALF + k * _L, _L)]
            cr_v[pl.ds(h * _HALF + k * _L, _L)] = plsc.load_gather(
                crows_v, [rid, cid])

    pltpu.sync_copy(le_v, le_out.at[pl.ds(base, _CHUNK)])
    pltpu.sync_copy(cr_v, cr_out.at[pl.ds(base, _CHUNK)])


_BLK = 2048


def _mlp_body(le_ref, ti_ref, cr_ref, tt_ref, w1a_ref, w1b_ref, b1_ref,
              w2_ref, b2_ref, w3_ref, b3_ref, out_ref):
    dot = functools.partial(
        lax.dot_general,
        dimension_numbers=(((1,), (0,)), ((), ())),
        precision=lax.Precision.HIGHEST,
    )
    ids = lax.broadcasted_iota(jnp.int32, (_BLK, _N_TIMES), 1)
    oh = (ids == ti_ref[...]).astype(jnp.float32)
    tw = dot(tt_ref[...], w1b_ref[...])
    h = dot(le_ref[...], w1a_ref[...]) + dot(oh, tw)
    h = jnp.maximum(h + b1_ref[...], 0.0)
    h = jnp.maximum(dot(h, w2_ref[...]) + b2_ref[...], 0.0)
    y = jnp.sum(h * w3_ref[...], axis=1, keepdims=True)
    out_ref[...] = y + b3_ref[...] + cr_ref[...]


@jax.jit
def _tc_mlp(le, ti, cr, tt, w1a, w1b, b1, w2, b2, w3r, b3):
    grid = _B // _BLK
    full = lambda i: (0, 0)
    return pl.pallas_call(
        _mlp_body,
        grid=(grid,),
        in_specs=[
            pl.BlockSpec((_BLK, _D_LINK), lambda i: (i, 0)),
            pl.BlockSpec((_BLK, 1), lambda i: (i, 0)),
            pl.BlockSpec((_BLK, 1), lambda i: (i, 0)),
            pl.BlockSpec((_N_TIMES, _D_TIME), full),
            pl.BlockSpec((_D_LINK, 128), full),
            pl.BlockSpec((_D_TIME, 128), full),
            pl.BlockSpec((1, 128), full),
            pl.BlockSpec((128, 64), full),
            pl.BlockSpec((1, 64), full),
            pl.BlockSpec((1, 64), full),
            pl.BlockSpec((1, 1), full),
        ],
        out_specs=pl.BlockSpec((_BLK, 1), lambda i: (i, 0)),
        out_shape=jax.ShapeDtypeStruct((_B, 1), jnp.float32),
    )(le, ti, cr, tt, w1a, w1b, b1, w2, b2, w3r, b3)


def kernel(link_idx, time_idx, link_table, time_table, cross_table,
           W1, b1, W2, b2, W3, b3):
    li = link_idx.astype(jnp.int32)
    ti = time_idx.astype(jnp.int32)
    le, cr = _sc_gather(li, ti,
                        link_table.reshape(-1, 128),
                        cross_table.reshape(-1, 128))
    y = _tc_mlp(
        le, ti[:, None], cr[:, None], time_table,
        W1[:_D_LINK], W1[_D_LINK:], b1[None, :],
        W2, b2[None, :], W3.reshape(1, 64), b3[None, :])
    return y[:, 0]


# speedup vs baseline: 1.5559x; 1.5559x over previous
import functools

import jax
import jax.numpy as jnp
from jax import lax
from jax.experimental import pallas as pl
from jax.experimental.pallas import tpu as pltpu
from jax.experimental.pallas import tpu_sc as plsc

_N_TIMES = 288
_B = 16384
_D_LINK = 32
_D_TIME = 8

_NC = 2
_NS = 16
_NW = _NC * _NS
_CHUNK = _B // _NW
_HALF = _CHUNK // 2
_L = 16

_sc_mesh = plsc.VectorSubcoreMesh(core_axis_name="c", subcore_axis_name="s")


@functools.partial(
    pl.kernel,
    mesh=_sc_mesh,
    compiler_params=pltpu.CompilerParams(
        use_tc_tiling_on_sc=True, needs_layout_passes=False),
    out_type=[
        jax.ShapeDtypeStruct((_B, _D_LINK), jnp.float32),
        jax.ShapeDtypeStruct((_B,), jnp.float32),
    ],
    scratch_types=[
        pltpu.VMEM((_CHUNK,), jnp.int32),
        pltpu.VMEM((_CHUNK,), jnp.int32),
        pltpu.VMEM((4, 128), jnp.int32),
        pltpu.VMEM((_CHUNK,), jnp.int32),
        pltpu.VMEM((128, 128), jnp.float32),
        pltpu.VMEM((128, 128), jnp.float32),
        pltpu.VMEM((_CHUNK, _D_LINK), jnp.float32),
        pltpu.VMEM((_CHUNK,), jnp.float32),
        pltpu.SemaphoreType.DMA,
        pltpu.SemaphoreType.DMA,
        pltpu.SemaphoreType.DMA,
    ],
)
def _sc_gather(link_idx_hbm, time_idx_hbm, link_tab_hbm, cross_tab_hbm,
               le_out, cr_out,
               li_v, ti_v, crow_v, ccol_v, crows0_v, crows1_v, le_v, cr_v,
               sem_l, sem_c0, sem_c1):
    wid = lax.axis_index("s") * _NC + lax.axis_index("c")
    base = wid * _CHUNK
    pltpu.sync_copy(link_idx_hbm.at[pl.ds(base, _CHUNK)], li_v)
    pltpu.sync_copy(time_idx_hbm.at[pl.ds(base, _CHUNK)], ti_v)

    for k in range(_CHUNK // _L):
        a = li_v[pl.ds(k * _L, _L)]
        b = ti_v[pl.ds(k * _L, _L)]
        ci = a * _N_TIMES + b
        g = k * _L
        crow_v[g // 128, pl.ds(g % 128, _L)] = lax.shift_right_logical(ci, 7)
        ccol_v[pl.ds(g, _L)] = lax.bitwise_and(ci, 127)

    bufs = (crows0_v, crows1_v)
    sems = (sem_c0, sem_c1)
    cross_copies = [
        pltpu.async_copy(cross_tab_hbm.at[crow_v.at[r]], bufs[r % 2],
                         sems[r % 2])
        for r in range(2)
    ]

    def _fire(c, carry):
        rv = li_v[pl.ds(c * _L, _L)]
        for j in range(_L):
            r = rv[j]
            pltpu.async_copy(
                link_tab_hbm.at[pl.ds(r, 1), :],
                le_v.at[pl.ds(c * _L + j, 1), :],
                sem_l)
        return carry

    lax.fori_loop(0, _CHUNK // _L, _fire, 0)

    for r in range(4):
        cross_copies[r].wait()
        buf = bufs[r % 2]
        for k in range(128 // _L):
            rid = lax.broadcasted_iota(jnp.int32, (_L,), 0) + k * _L
            cid = ccol_v[pl.ds(r * 128 + k * _L, _L)]
            cr_v[pl.ds(r * 128 + k * _L, _L)] = plsc.load_gather(
                buf, [rid, cid])
        if r + 2 < 4:
            cross_copies.append(
                pltpu.async_copy(cross_tab_hbm.at[crow_v.at[r + 2]],
                                 bufs[r % 2], sems[r % 2]))

    def _drain(c, carry):
        pltpu.make_async_copy(
            link_tab_hbm.at[pl.ds(0, 1), :],
            le_v.at[pl.ds(0, 1), :],
            sem_l).wait()
        return carry

    lax.fori_loop(0, _CHUNK, _drain, 0)
    pltpu.sync_copy(le_v, le_out.at[pl.ds(base, _CHUNK)])
    pltpu.sync_copy(cr_v, cr_out.at[pl.ds(base, _CHUNK)])


_GRID = 16
_BLK = _B // _GRID


def _mlp_body(le_ref, ti_ref, cr_ref, ttT_ref, w1aT_ref, w1bT_ref, b1_ref,
              w2T_ref, b2_ref, w3T_ref, b3_ref, out_ref):
    dot = functools.partial(
        lax.dot_general,
        dimension_numbers=(((1,), (0,)), ((), ())),
        precision=lax.Precision.HIGHEST,
    )
    leT = lax.transpose(le_ref[...], (1, 0))
    twT = dot(w1bT_ref[...], ttT_ref[...])
    ids = lax.broadcasted_iota(jnp.int32, (_N_TIMES, _BLK), 0)
    oh = (ids == ti_ref[0]).astype(jnp.float32)
    h = dot(w1aT_ref[...], leT) + dot(twT, oh)
    h = jnp.maximum(h + b1_ref[...], 0.0)
    h = jnp.maximum(dot(w2T_ref[...], h) + b2_ref[...], 0.0)
    y = dot(w3T_ref[...], h)
    out_ref[...] = (y + b3_ref[...] + cr_ref[0])[None]


@jax.jit
def _tc_mlp(le, ti2, cr2, ttT, w1aT, w1bT, b1c, w2T, b2c, w3T, b3c):
    full = lambda i: (0, 0)
    return pl.pallas_call(
        _mlp_body,
        grid=(_GRID,),
        in_specs=[
            pl.BlockSpec((_BLK, _D_LINK), lambda i: (i, 0)),
            pl.BlockSpec((1, 1, _BLK), lambda i: (i, 0, 0)),
            pl.BlockSpec((1, 1, _BLK), lambda i: (i, 0, 0)),
            pl.BlockSpec((_D_TIME, _N_TIMES), full),
            pl.BlockSpec((128, _D_LINK), full),
            pl.BlockSpec((128, _D_TIME), full),
            pl.BlockSpec((128, 1), full),
            pl.BlockSpec((64, 128), full),
            pl.BlockSpec((64, 1), full),
            pl.BlockSpec((1, 64), full),
            pl.BlockSpec((1, 1), full),
        ],
        out_specs=pl.BlockSpec((1, 1, _BLK), lambda i: (i, 0, 0)),
        out_shape=jax.ShapeDtypeStruct((_GRID, 1, _BLK), jnp.float32),
    )(le, ti2, cr2, ttT, w1aT, w1bT, b1c, w2T, b2c, w3T, b3c)


def kernel(link_idx, time_idx, link_table, time_table, cross_table,
           W1, b1, W2, b2, W3, b3):
    li = link_idx.astype(jnp.int32)
    ti = time_idx.astype(jnp.int32)
    le, cr = _sc_gather(li, ti, link_table, cross_table.reshape(-1, 128))
    y = _tc_mlp(
        le, ti.reshape(_GRID, 1, _BLK), cr.reshape(_GRID, 1, _BLK),
        time_table.T, W1[:_D_LINK].T, W1[_D_LINK:].T, b1[:, None],
        W2.T, b2[:, None], W3.reshape(1, 64), b3[None, :])
    return y.reshape(_B)


# speedup vs baseline: 1.5905x; 1.0222x over previous
import functools

import jax
import jax.numpy as jnp
from jax import lax
from jax.experimental import pallas as pl
from jax.experimental.pallas import tpu as pltpu
from jax.experimental.pallas import tpu_sc as plsc

_N_TIMES = 288
_B = 16384
_D_LINK = 32
_D_TIME = 8

_NC = 2
_NS = 16
_NW = _NC * _NS
_CHUNK = _B // _NW
_L = 16

_sc_mesh = plsc.VectorSubcoreMesh(core_axis_name="c", subcore_axis_name="s")


@functools.partial(
    pl.kernel,
    mesh=_sc_mesh,
    compiler_params=pltpu.CompilerParams(
        use_tc_tiling_on_sc=True, needs_layout_passes=False),
    out_type=[
        jax.ShapeDtypeStruct((_B, _D_LINK), jnp.float32),
        jax.ShapeDtypeStruct((_B * (_D_TIME + 1),), jnp.float32),
    ],
    scratch_types=[
        pltpu.VMEM((_CHUNK,), jnp.int32),
        pltpu.VMEM((_CHUNK,), jnp.int32),
        pltpu.VMEM((4, 128), jnp.int32),
        pltpu.VMEM((_CHUNK,), jnp.int32),
        pltpu.VMEM((128, 128), jnp.float32),
        pltpu.VMEM((128, 128), jnp.float32),
        pltpu.VMEM((_CHUNK, _D_LINK), jnp.float32),
        pltpu.VMEM((_N_TIMES * _D_TIME,), jnp.float32),
        pltpu.VMEM((_CHUNK * _D_TIME,), jnp.float32),
        pltpu.VMEM((_CHUNK,), jnp.float32),
        pltpu.SemaphoreType.DMA,
        pltpu.SemaphoreType.DMA,
    ],
)
def _sc_gather(link_idx_hbm, time_idx_hbm, link_tab_hbm, time_tab_hbm,
               cross_tab_hbm, le_out, tecr_out,
               li_v, ti_v, crow_v, ccol_v, crows0_v, crows1_v, le_v,
               tt_v, te_v, cr_v, sem_l, sem_c):
    wid = lax.axis_index("s") * _NC + lax.axis_index("c")
    base = wid * _CHUNK
    pltpu.sync_copy(link_idx_hbm.at[pl.ds(base, _CHUNK)], li_v)
    pltpu.sync_copy(time_idx_hbm.at[pl.ds(base, _CHUNK)], ti_v)
    pltpu.sync_copy(time_tab_hbm, tt_v)

    for k in range(_CHUNK // _L):
        a = li_v[pl.ds(k * _L, _L)]
        b = ti_v[pl.ds(k * _L, _L)]
        ci = a * _N_TIMES + b
        g = k * _L
        crow_v[g // 128, pl.ds(g % 128, _L)] = lax.shift_right_logical(ci, 7)
        ccol_v[pl.ds(g, _L)] = lax.bitwise_and(ci, 127)

    bufs = (crows0_v, crows1_v)
    cross_copies = [
        pltpu.async_copy(cross_tab_hbm.at[crow_v.at[r]], bufs[r % 2], sem_c)
        for r in range(2)
    ]

    def _fire(c, carry):
        rv = li_v[pl.ds(c * _L, _L)]
        for j in range(_L):
            pltpu.async_copy(
                link_tab_hbm.at[pl.ds(rv[j], 1), :],
                le_v.at[pl.ds(c * _L + j, 1), :], sem_l)
        return carry

    lax.fori_loop(0, _CHUNK // _L, _fire, 0)

    def _te(c, carry):
        f = lax.broadcasted_iota(jnp.int32, (_L,), 0) + c * _L
        rows = lax.shift_right_logical(f, 3)
        cols = lax.bitwise_and(f, 7)
        tirow = plsc.load_gather(ti_v, [rows])
        te_v[pl.ds(c * _L, _L)] = plsc.load_gather(
            tt_v, [tirow * _D_TIME + cols])
        return carry

    lax.fori_loop(0, _CHUNK * _D_TIME // _L, _te, 0)

    for r in range(4):
        cross_copies[r].wait()
        buf = bufs[r % 2]
        for k in range(128 // _L):
            rid = lax.broadcasted_iota(jnp.int32, (_L,), 0) + k * _L
            cid = ccol_v[pl.ds(r * 128 + k * _L, _L)]
            cr_v[pl.ds(r * 128 + k * _L, _L)] = plsc.load_gather(
                buf, [rid, cid])
        if r + 2 < 4:
            cross_copies.append(
                pltpu.async_copy(cross_tab_hbm.at[crow_v.at[r + 2]],
                                 bufs[r % 2], sem_c))

    def _drain(c, carry):
        pltpu.make_async_copy(
            link_tab_hbm.at[pl.ds(0, 1), :],
            le_v.at[pl.ds(0, 1), :], sem_l).wait()
        return carry

    lax.fori_loop(0, _CHUNK, _drain, 0)
    pltpu.sync_copy(le_v, le_out.at[pl.ds(base, _CHUNK)])
    pltpu.sync_copy(te_v, tecr_out.at[pl.ds(base * _D_TIME,
                                            _CHUNK * _D_TIME)])
    pltpu.sync_copy(cr_v, tecr_out.at[pl.ds(_B * _D_TIME + base, _CHUNK)])


_GRID = 8
_BLK = _B // _GRID


def _mlp_body(le_ref, te_ref, cr_ref, w1aT_ref, w1bT_ref, b1_ref,
              w2T_ref, b2_ref, w3T_ref, b3_ref, out_ref):
    dot = functools.partial(
        lax.dot_general,
        dimension_numbers=(((1,), (0,)), ((), ())),
        precision=lax.Precision.HIGHEST,
    )
    leT = lax.transpose(le_ref[...], (1, 0))
    teT = lax.transpose(te_ref[...], (1, 0))
    h = dot(w1aT_ref[...], leT) + dot(w1bT_ref[...], teT)
    h = jnp.maximum(h + b1_ref[...], 0.0)
    h = jnp.maximum(dot(w2T_ref[...], h) + b2_ref[...], 0.0)
    y = dot(w3T_ref[...], h)
    out_ref[...] = (y + b3_ref[...] + cr_ref[0])[None]


@jax.jit
def _tc_mlp(le, te, cr2, w1aT, w1bT, b1c, w2T, b2c, w3T, b3c):
    full = lambda i: (0, 0)
    return pl.pallas_call(
        _mlp_body,
        grid=(_GRID,),
        in_specs=[
            pl.BlockSpec((_BLK, _D_LINK), lambda i: (i, 0)),
            pl.BlockSpec((_BLK, _D_TIME), lambda i: (i, 0)),
            pl.BlockSpec((1, 1, _BLK), lambda i: (i, 0, 0)),
            pl.BlockSpec((128, _D_LINK), full),
            pl.BlockSpec((128, _D_TIME), full),
            pl.BlockSpec((128, 1), full),
            pl.BlockSpec((64, 128), full),
            pl.BlockSpec((64, 1), full),
            pl.BlockSpec((1, 64), full),
            pl.BlockSpec((1, 1), full),
        ],
        out_specs=pl.BlockSpec((1, 1, _BLK), lambda i: (i, 0, 0)),
        out_shape=jax.ShapeDtypeStruct((_GRID, 1, _BLK), jnp.float32),
    )(le, te, cr2, w1aT, w1bT, b1c, w2T, b2c, w3T, b3c)


def kernel(link_idx, time_idx, link_table, time_table, cross_table,
           W1, b1, W2, b2, W3, b3):
    li = link_idx.astype(jnp.int32)
    ti = time_idx.astype(jnp.int32)
    le, tecr = _sc_gather(li, ti, link_table, time_table.reshape(-1),
                          cross_table.reshape(-1, 128))
    te = tecr[:_B * _D_TIME].reshape(_B, _D_TIME)
    cr = tecr[_B * _D_TIME:]
    y = _tc_mlp(
        le, te, cr.reshape(_GRID, 1, _BLK),
        W1[:_D_LINK].T, W1[_D_LINK:].T, b1[:, None],
        W2.T, b2[:, None], W3.reshape(1, 64), b3[None, :])
    return y.reshape(_B)
